# Initial kernel scaffold; baseline (speedup 1.0000x reference)
import functools

import jax
import jax.numpy as jnp
from jax import lax
from jax.experimental import pallas as pl
from jax.experimental.pallas import tpu as pltpu
from jax.experimental.pallas import tpu_sc as plsc

N = 10000
E = 320000
D = 128
ALPHA = 0.5
BETA = 1.0

NC = 2
NS = 16
NW = NC * NS
CHUNK = 128
CH = 80
EPAD = NW * CH * CHUNK
NPAD = 10016
ROWS_PER_TILE = NPAD // NS
NBLK = 8
BLK = NPAD // NBLK


def _sc_deg_body(dst_hbm, zrow_hbm, out_hbm, dst_v, ones_v, deg_sh):
    c = lax.axis_index("c")
    s = lax.axis_index("s")
    wid = c * NS + s

    @pl.when(s == 0)
    def _zero():
        pltpu.sync_copy(zrow_hbm, deg_sh)

    for k in range(CHUNK // 16):
        ones_v[pl.ds(k * 16, 16)] = jnp.full((16,), 1.0, jnp.float32)
    pltpu.sync_copy(dst_hbm.at[wid], dst_v)
    plsc.subcore_barrier()

    def body(ch, carry):
        pltpu.sync_copy(ones_v, deg_sh.at[dst_v.at[ch]], add=True)
        return carry

    lax.fori_loop(0, CH, body, 0)
    plsc.subcore_barrier()

    @pl.when(s == 0)
    def _dump():
        pltpu.sync_copy(deg_sh, out_hbm.at[c])


def _sc_agg_body(ht_hbm, src_hbm, dst_hbm, z2_hbm, out_hbm,
                 src_v, dst_v, rows_v, acc_sh, sem0, sem1):
    c = lax.axis_index("c")
    s = lax.axis_index("s")
    wid = c * NS + s
    base = s * ROWS_PER_TILE

    pltpu.sync_copy(z2_hbm.at[pl.ds(base, ROWS_PER_TILE)],
                    acc_sh.at[pl.ds(base, ROWS_PER_TILE)])
    pltpu.sync_copy(src_hbm.at[wid], src_v)
    pltpu.sync_copy(dst_hbm.at[wid], dst_v)
    plsc.subcore_barrier()

    pltpu.make_async_copy(ht_hbm.at[src_v.at[0]], rows_v.at[0], sem0).start()

    def body(gp, carry):
        ch0 = 2 * gp
        ch1 = ch0 + 1
        pltpu.make_async_copy(ht_hbm.at[src_v.at[ch1]], rows_v.at[1],
                              sem1).start()
        pltpu.make_async_copy(ht_hbm.at[src_v.at[ch0]], rows_v.at[0],
                              sem0).wait()
        pltpu.sync_copy(rows_v.at[0], acc_sh.at[dst_v.at[ch0]], add=True)

        @pl.when(ch0 + 2 < CH)
        def _next():
            pltpu.make_async_copy(ht_hbm.at[src_v.at[ch0 + 2]], rows_v.at[0],
                                  sem0).start()

        pltpu.make_async_copy(ht_hbm.at[src_v.at[ch1]], rows_v.at[1],
                              sem1).wait()
        pltpu.sync_copy(rows_v.at[1], acc_sh.at[dst_v.at[ch1]], add=True)
        return carry

    lax.fori_loop(0, CH // 2, body, 0)
    plsc.subcore_barrier()
    pltpu.sync_copy(acc_sh.at[pl.ds(base, ROWS_PER_TILE)],
                    out_hbm.at[c, pl.ds(base, ROWS_PER_TILE)])


_sc_mesh = plsc.VectorSubcoreMesh(core_axis_name="c", subcore_axis_name="s")

_sc_deg = functools.partial(
    pl.kernel,
    mesh=_sc_mesh,
    out_type=jax.ShapeDtypeStruct((NC, NPAD), jnp.float32),
    scratch_types=[
        pltpu.VMEM((CH, CHUNK), jnp.int32),
        pltpu.VMEM((CHUNK,), jnp.float32),
        pltpu.VMEM_SHARED((NPAD,), jnp.float32),
    ],
)(_sc_deg_body)

_sc_agg = functools.partial(
    pl.kernel,
    mesh=_sc_mesh,
    out_type=jax.ShapeDtypeStruct((NC, NPAD, D), jnp.float32),
    scratch_types=[
        pltpu.VMEM((CH, CHUNK), jnp.int32),
        pltpu.VMEM((CH, CHUNK), jnp.int32),
        pltpu.VMEM((2, CHUNK, D), jnp.float32),
        pltpu.VMEM_SHARED((NPAD, D), jnp.float32),
        pltpu.SemaphoreType.DMA,
        pltpu.SemaphoreType.DMA,
    ],
)(_sc_agg_body)


def _tc_lin1_body(x_ref, w1t_ref, b1_ref, degp_ref, h_ref, ht_ref):
    i = pl.program_id(0)
    h = jnp.dot(x_ref[...], w1t_ref[...],
                preferred_element_type=jnp.float32) + b1_ref[...]
    deg = degp_ref[0] + degp_ref[1] + BETA
    dinv = lax.rsqrt(deg)
    rows = i * BLK + lax.broadcasted_iota(jnp.int32, (BLK, 1), 0)
    ht = jnp.where(rows < N, h * dinv[:, None], 0.0)
    h_ref[...] = h
    ht_ref[...] = ht


def _tc_out_body(h_ref, p_ref, degp_ref, w2t_ref, b2_ref, o_ref):
    h = h_ref[...]
    sagg = p_ref[0] + p_ref[1]
    deg = degp_ref[0] + degp_ref[1] + BETA
    dinv = lax.rsqrt(deg)
    agg = dinv[:, None] * sagg + (BETA * (dinv * dinv))[:, None] * h
    o = ALPHA * h + (1.0 - ALPHA) * agg
    nrm = jnp.sqrt(jnp.sum(o * o, axis=1, keepdims=True))
    o = o / jnp.maximum(nrm, 1e-12)
    o = jnp.maximum(o, 0.0)
    o_ref[...] = jnp.dot(o, w2t_ref[...],
                         preferred_element_type=jnp.float32) + b2_ref[...]


_tc_lin1 = pl.pallas_call(
    _tc_lin1_body,
    grid=(NBLK,),
    in_specs=[
        pl.BlockSpec((BLK, D), lambda i: (i, 0)),
        pl.BlockSpec((D, D), lambda i: (0, 0)),
        pl.BlockSpec((1, D), lambda i: (0, 0)),
        pl.BlockSpec((2, BLK), lambda i: (0, i)),
    ],
    out_specs=[
        pl.BlockSpec((BLK, D), lambda i: (i, 0)),
        pl.BlockSpec((BLK, D), lambda i: (i, 0)),
    ],
    out_shape=[
        jax.ShapeDtypeStruct((NPAD, D), jnp.float32),
        jax.ShapeDtypeStruct((NPAD, D), jnp.float32),
    ],
)

_tc_out = pl.pallas_call(
    _tc_out_body,
    grid=(NBLK,),
    in_specs=[
        pl.BlockSpec((BLK, D), lambda i: (i, 0)),
        pl.BlockSpec((NC, BLK, D), lambda i: (0, i, 0)),
        pl.BlockSpec((2, BLK), lambda i: (0, i)),
        pl.BlockSpec((D, D), lambda i: (0, 0)),
        pl.BlockSpec((1, D), lambda i: (0, 0)),
    ],
    out_specs=pl.BlockSpec((BLK, D), lambda i: (i, 0)),
    out_shape=jax.ShapeDtypeStruct((NPAD, D), jnp.float32),
)


@jax.jit
def kernel(x, edge_index, W1, b1, W2, b2):
    src = edge_index[0]
    dst = edge_index[1]
    npad_rows = NPAD - N
    pad_idx = N + (jnp.arange(EPAD - E, dtype=jnp.int32) % npad_rows)
    src_p = jnp.concatenate([src, pad_idx]).reshape(NW, CH, CHUNK)
    dst_p = jnp.concatenate([dst, pad_idx]).reshape(NW, CH, CHUNK)
    x_p = jnp.pad(x, ((0, npad_rows), (0, 0)))
    zrow = jnp.zeros((NPAD,), jnp.float32)
    z2 = jnp.zeros((NPAD, D), jnp.float32)

    degp = _sc_deg(dst_p, zrow)
    h, ht = _tc_lin1(x_p, W1.T, b1[None, :], degp)
    aggp = _sc_agg(ht, src_p, dst_p, z2)
    out = _tc_out(h, aggp, degp, W2.T, b2[None, :])
    return out[:N]


# speedup vs baseline: 36.3252x; 36.3252x over previous
import functools

import jax
import jax.numpy as jnp
from jax import lax
from jax.experimental import pallas as pl
from jax.experimental.pallas import tpu as pltpu
from jax.experimental.pallas import tpu_sc as plsc

N = 10000
E = 320000
D = 128
ALPHA = 0.5
BETA = 1.0

NC = 2
NS = 16
NW = NC * NS
CHUNK = 128
CH = 80
HCH = CH // 2
EPAD = NW * CH * CHUNK
NPAD = 10240
ROWS_PER_TILE = NPAD // NS
NBLK = 8
BLK = NPAD // NBLK


def _sc_deg_body(dst_hbm, zrow_hbm, out_hbm, dst_v, ones_v, deg_sh):
    c = lax.axis_index("c")
    s = lax.axis_index("s")
    wid = c * NS + s

    @pl.when(s == 0)
    def _zero():
        pltpu.sync_copy(zrow_hbm, deg_sh)

    for k in range(CHUNK // 16):
        ones_v[pl.ds(k * 16, 16)] = jnp.full((16,), 1.0, jnp.float32)
    pltpu.sync_copy(dst_hbm.at[wid], dst_v)
    plsc.subcore_barrier()

    def body(ch, carry):
        pltpu.sync_copy(ones_v, deg_sh.at[dst_v.at[ch]], add=True)
        return carry

    lax.fori_loop(0, CH, body, 0)
    plsc.subcore_barrier()

    @pl.when(s == 0)
    def _dump():
        pltpu.sync_copy(deg_sh, out_hbm.at[c])


def _sc_agg_body(ht_hbm, src_hbm, dst_hbm, z2_hbm, out_hbm,
                 src_v, dst_v, rows_v, acc_sh, sem0, sem1):
    c = lax.axis_index("c")
    s = lax.axis_index("s")
    wid = c * NS + s
    base = s * ROWS_PER_TILE

    pltpu.sync_copy(z2_hbm.at[pl.ds(base, ROWS_PER_TILE)],
                    acc_sh.at[pl.ds(base, ROWS_PER_TILE)])
    plsc.subcore_barrier()

    for hh in range(CH // HCH):
        pltpu.sync_copy(src_hbm.at[wid, pl.ds(hh * HCH, HCH)], src_v)
        pltpu.sync_copy(dst_hbm.at[wid, pl.ds(hh * HCH, HCH)], dst_v)
        pltpu.make_async_copy(ht_hbm.at[src_v.at[0]], rows_v.at[0],
                              sem0).start()

        def body(gp, carry):
            ch0 = 2 * gp
            ch1 = ch0 + 1
            pltpu.make_async_copy(ht_hbm.at[src_v.at[ch1]], rows_v.at[1],
                                  sem1).start()
            pltpu.make_async_copy(ht_hbm.at[src_v.at[ch0]], rows_v.at[0],
                                  sem0).wait()
            pltpu.sync_copy(rows_v.at[0], acc_sh.at[dst_v.at[ch0]], add=True)

            @pl.when(ch0 + 2 < HCH)
            def _next():
                pltpu.make_async_copy(ht_hbm.at[src_v.at[ch0 + 2]],
                                      rows_v.at[0], sem0).start()

            pltpu.make_async_copy(ht_hbm.at[src_v.at[ch1]], rows_v.at[1],
                                  sem1).wait()
            pltpu.sync_copy(rows_v.at[1], acc_sh.at[dst_v.at[ch1]], add=True)
            return carry

        lax.fori_loop(0, HCH // 2, body, 0)
    plsc.subcore_barrier()
    pltpu.sync_copy(acc_sh.at[pl.ds(base, ROWS_PER_TILE)],
                    out_hbm.at[c, pl.ds(base, ROWS_PER_TILE)])


_sc_mesh = plsc.VectorSubcoreMesh(core_axis_name="c", subcore_axis_name="s")

_sc_deg = functools.partial(
    pl.kernel,
    mesh=_sc_mesh,
    out_type=jax.ShapeDtypeStruct((NC, NPAD), jnp.float32),
    scratch_types=[
        pltpu.VMEM((CH, CHUNK), jnp.int32),
        pltpu.VMEM((CHUNK,), jnp.float32),
        pltpu.VMEM_SHARED((NPAD,), jnp.float32),
    ],
)(_sc_deg_body)

_sc_agg = functools.partial(
    pl.kernel,
    mesh=_sc_mesh,
    out_type=jax.ShapeDtypeStruct((NC, NPAD, D), jnp.float32),
    scratch_types=[
        pltpu.VMEM((HCH, CHUNK), jnp.int32),
        pltpu.VMEM((HCH, CHUNK), jnp.int32),
        pltpu.VMEM((2, CHUNK, D), jnp.float32),
        pltpu.VMEM_SHARED((NPAD, D), jnp.float32),
        pltpu.SemaphoreType.DMA,
        pltpu.SemaphoreType.DMA,
    ],
)(_sc_agg_body)


def _tc_lin1_body(x_ref, w1t_ref, b1_ref, degp_ref, h_ref, ht_ref):
    i = pl.program_id(0)
    h = jnp.dot(x_ref[...], w1t_ref[...],
                preferred_element_type=jnp.float32) + b1_ref[...]
    deg = degp_ref[0] + degp_ref[1] + BETA
    dinv = lax.rsqrt(deg)
    rows = i * BLK + lax.broadcasted_iota(jnp.int32, (BLK, 1), 0)
    ht = jnp.where(rows < N, h * dinv[:, None], 0.0)
    h_ref[...] = h
    ht_ref[...] = ht


def _tc_out_body(h_ref, p_ref, degp_ref, w2t_ref, b2_ref, o_ref):
    h = h_ref[...]
    sagg = p_ref[0] + p_ref[1]
    deg = degp_ref[0] + degp_ref[1] + BETA
    dinv = lax.rsqrt(deg)
    agg = dinv[:, None] * sagg + (BETA * (dinv * dinv))[:, None] * h
    o = ALPHA * h + (1.0 - ALPHA) * agg
    nrm = jnp.sqrt(jnp.sum(o * o, axis=1, keepdims=True))
    o = o / jnp.maximum(nrm, 1e-12)
    o = jnp.maximum(o, 0.0)
    o_ref[...] = jnp.dot(o, w2t_ref[...],
                         preferred_element_type=jnp.float32) + b2_ref[...]


_tc_lin1 = pl.pallas_call(
    _tc_lin1_body,
    grid=(NBLK,),
    in_specs=[
        pl.BlockSpec((BLK, D), lambda i: (i, 0)),
        pl.BlockSpec((D, D), lambda i: (0, 0)),
        pl.BlockSpec((1, D), lambda i: (0, 0)),
        pl.BlockSpec((2, BLK), lambda i: (0, i)),
    ],
    out_specs=[
        pl.BlockSpec((BLK, D), lambda i: (i, 0)),
        pl.BlockSpec((BLK, D), lambda i: (i, 0)),
    ],
    out_shape=[
        jax.ShapeDtypeStruct((NPAD, D), jnp.float32),
        jax.ShapeDtypeStruct((NPAD, D), jnp.float32),
    ],
)

_tc_out = pl.pallas_call(
    _tc_out_body,
    grid=(NBLK,),
    in_specs=[
        pl.BlockSpec((BLK, D), lambda i: (i, 0)),
        pl.BlockSpec((NC, BLK, D), lambda i: (0, i, 0)),
        pl.BlockSpec((2, BLK), lambda i: (0, i)),
        pl.BlockSpec((D, D), lambda i: (0, 0)),
        pl.BlockSpec((1, D), lambda i: (0, 0)),
    ],
    out_specs=pl.BlockSpec((BLK, D), lambda i: (i, 0)),
    out_shape=jax.ShapeDtypeStruct((NPAD, D), jnp.float32),
)


@jax.jit
def kernel(x, edge_index, W1, b1, W2, b2):
    src = edge_index[0]
    dst = edge_index[1]
    npad_rows = NPAD - N
    pad_idx = N + (jnp.arange(EPAD - E, dtype=jnp.int32) % npad_rows)
    src_p = jnp.concatenate([src, pad_idx]).reshape(NW, CH, CHUNK)
    dst_p = jnp.concatenate([dst, pad_idx]).reshape(NW, CH, CHUNK)
    x_p = jnp.pad(x, ((0, npad_rows), (0, 0)))
    zrow = jnp.zeros((NPAD,), jnp.float32)
    z2 = jnp.zeros((NPAD, D), jnp.float32)

    degp = _sc_deg(dst_p, zrow)
    h, ht = _tc_lin1(x_p, W1.T, b1[None, :], degp)
    aggp = _sc_agg(ht, src_p, dst_p, z2)
    out = _tc_out(h, aggp, degp, W2.T, b2[None, :])
    return out[:N]
